# baseline (device time: 15507 ns/iter reference)
import jax
import jax.numpy as jnp
from jax import lax
from jax.experimental import pallas as pl
from jax.experimental.pallas import tpu as pltpu

H = 2


def kernel(t, W):
    m, k = t.shape
    _, n = W.shape
    mh = m // H
    q = mh // 2

    def body(t_ref, w_ref, out_ref, c0, c1, c2, send_sems, recv_sems):
        my = lax.axis_index("i")
        peer1 = my ^ 1
        peer2 = 3 - my

        barrier_sem = pltpu.get_barrier_semaphore()
        for nbr in (peer1, peer2):
            pl.semaphore_signal(
                barrier_sem, inc=1,
                device_id=(nbr,), device_id_type=pl.DeviceIdType.MESH,
            )

        kA0 = jnp.logical_or(my == 0, my == 3)
        keep = [jnp.where(kA0, 0, q), mh + jnp.where(my < 2, 0, q)]
        give = [jnp.where(kA0, q, 0), mh + jnp.where(my < 2, q, 0)]
        stage_peer = [(peer1, peer2, peer1), (peer2, peer1, peer2)]

        def rdma(stage, h, src_ref, dst_ref, off):
            return pltpu.make_async_remote_copy(
                src_ref=src_ref.at[pl.ds(off, q), :],
                dst_ref=dst_ref.at[pl.ds(off, q), :],
                send_sem=send_sems.at[stage, h],
                recv_sem=recv_sems.at[stage, h],
                device_id=(stage_peer[h][stage],),
                device_id_type=pl.DeviceIdType.MESH,
            )

        def mm(off):
            rows = pl.ds(off, q)
            out_ref[rows, :] = jnp.dot(
                t_ref[rows, :], w_ref[...], preferred_element_type=jnp.float32
            )

        mm(give[0])
        pl.semaphore_wait(barrier_sem, 2)
        s1 = [rdma(0, 0, out_ref, c0, give[0])]
        s1[0].start()
        mm(give[1])
        s1.append(rdma(0, 1, out_ref, c0, give[1]))
        s1[1].start()
        mm(keep[0])
        mm(keep[1])

        s2 = []
        for h in range(H):
            rows = pl.ds(keep[h], q)
            s1[h].wait_recv()
            c0[rows, :] += out_ref[rows, :]
            r = rdma(1, h, c0, c1, keep[h])
            r.start()
            s2.append(r)

        s3 = []
        for h in range(H):
            rows = pl.ds(keep[h], q)
            s2[h].wait_recv()
            c1[rows, :] += c0[rows, :]
            r = rdma(2, h, c1, c2, keep[h])
            r.start()
            s3.append(r)
            out_ref[rows, :] = c1[rows, :]

        for h in range(H):
            rows = pl.ds(give[h], q)
            s3[h].wait_recv()
            s1[h].wait_send()
            out_ref[rows, :] = c2[rows, :]

        for h in range(H):
            s2[h].wait_send()
            s3[h].wait_send()

    return pl.pallas_call(
        body,
        out_shape=jax.ShapeDtypeStruct((m, n), jnp.float32),
        in_specs=[
            pl.BlockSpec(memory_space=pltpu.VMEM),
            pl.BlockSpec(memory_space=pltpu.VMEM),
        ],
        out_specs=pl.BlockSpec(memory_space=pltpu.VMEM),
        scratch_shapes=[
            pltpu.VMEM((m, n), jnp.float32),
            pltpu.VMEM((m, n), jnp.float32),
            pltpu.VMEM((m, n), jnp.float32),
            pltpu.SemaphoreType.DMA((3, H)),
            pltpu.SemaphoreType.DMA((3, H)),
        ],
        compiler_params=pltpu.CompilerParams(collective_id=0),
    )(t, W)


# device time: 13644 ns/iter; 1.1365x vs baseline; 1.1365x over previous
import jax
import jax.numpy as jnp
from jax import lax
from jax.experimental import pallas as pl
from jax.experimental.pallas import tpu as pltpu

H = 2
C2 = 2


def kernel(t, W):
    m, k = t.shape
    _, n = W.shape
    mh = m // H
    mc = mh // C2

    def body(t_ref, w_ref, out_ref, comm_ref, send_sems, recv_sems):
        my = lax.axis_index("i")
        peer1 = my ^ 1
        peer2 = 3 - my
        peers = (peer1, peer2)

        barrier_sem = pltpu.get_barrier_semaphore()
        for nbr in peers:
            pl.semaphore_signal(
                barrier_sem, inc=1,
                device_id=(nbr,), device_id_type=pl.DeviceIdType.MESH,
            )

        chunks = [
            (h, c, h * mh + c * mc) for c in range(C2) for h in range(H)
        ]

        def exchange(stage, h, c, row, peer):
            return pltpu.make_async_remote_copy(
                src_ref=out_ref.at[pl.ds(row, mc), :],
                dst_ref=comm_ref.at[stage, pl.ds(row, mc), :],
                send_sem=send_sems.at[stage, h, c],
                recv_sem=recv_sems.at[stage, h, c],
                device_id=(peer,),
                device_id_type=pl.DeviceIdType.MESH,
            )

        s1 = {}
        for i, (h, c, row) in enumerate(chunks):
            rows = pl.ds(row, mc)
            out_ref[rows, :] = jnp.dot(
                t_ref[rows, :], w_ref[...], preferred_element_type=jnp.float32
            )
            if i == 0:
                pl.semaphore_wait(barrier_sem, 2)
            r = exchange(0, h, c, row, peers[h])
            r.start()
            s1[(h, c)] = r

        s2 = {}
        for h, c, row in chunks:
            rows = pl.ds(row, mc)
            s1[(h, c)].wait_recv()
            comm_ref[0, rows, :] += out_ref[rows, :]
            r = pltpu.make_async_remote_copy(
                src_ref=comm_ref.at[0, pl.ds(row, mc), :],
                dst_ref=comm_ref.at[1, pl.ds(row, mc), :],
                send_sem=send_sems.at[1, h, c],
                recv_sem=recv_sems.at[1, h, c],
                device_id=(peers[1 - h],),
                device_id_type=pl.DeviceIdType.MESH,
            )
            r.start()
            s2[(h, c)] = r

        for h, c, row in chunks:
            rows = pl.ds(row, mc)
            s2[(h, c)].wait_recv()
            s1[(h, c)].wait_send()
            out_ref[rows, :] = comm_ref[0, rows, :] + comm_ref[1, rows, :]
        for h, c, row in chunks:
            s2[(h, c)].wait_send()

    return pl.pallas_call(
        body,
        out_shape=jax.ShapeDtypeStruct((m, n), jnp.float32),
        in_specs=[
            pl.BlockSpec(memory_space=pltpu.VMEM),
            pl.BlockSpec(memory_space=pltpu.VMEM),
        ],
        out_specs=pl.BlockSpec(memory_space=pltpu.VMEM),
        scratch_shapes=[
            pltpu.VMEM((2, m, n), jnp.float32),
            pltpu.SemaphoreType.DMA((2, H, C2)),
            pltpu.SemaphoreType.DMA((2, H, C2)),
        ],
        compiler_params=pltpu.CompilerParams(collective_id=0),
    )(t, W)
